# baseline (device time: 199960 ns/iter reference)
import jax
import jax.numpy as jnp
from jax import lax
from jax.experimental import pallas as pl
from jax.experimental.pallas import tpu as pltpu

N_DEV = 8
B = 2
SQL = 512
DM = 768
SKV = 512
HQ = 64
DH = 64
H_LOC = HQ // N_DEV
HD_LOC = H_LOC * DH
QBLK = 64
VA = 128


def kernel(x, Wq, K_ext, V_ext, Wo):
    xb = (x * 0.125).astype(jnp.bfloat16)
    wqb = Wq.astype(jnp.bfloat16)
    wob = Wo.astype(jnp.bfloat16)

    def body(x_ref, wq_ref, k_ref, v_ref, wo_ref, out_ref,
             wq_buf, wo_buf, bias_ref, ctx_ref, kscr, vscr, vaug_ref,
             send_sems, recv_sems, dma_sems):
        my = lax.axis_index("i")
        left = lax.rem(my - 1 + N_DEV, N_DEV)
        right = lax.rem(my + 1, N_DEV)

        barrier = pltpu.get_barrier_semaphore()
        for nbr in (left, right):
            pl.semaphore_signal(barrier, inc=1, device_id=(nbr,),
                                device_id_type=pl.DeviceIdType.MESH)
        pl.semaphore_wait(barrier, 2)

        row = lax.broadcasted_iota(jnp.int32, (SQL, SKV), 0)
        col = lax.broadcasted_iota(jnp.int32, (SQL, SKV), 1)
        qb = my * (SQL // QBLK) + row // QBLK
        kb = col // QBLK
        keep = (qb == kb) | (kb == 0) | (lax.rem(qb + kb, 3) == 0)
        bias_ref[...] = jnp.where(keep, 0.0, -1e9).astype(jnp.float32)

        acol = lax.broadcasted_iota(jnp.int32, (SQL, VA), 1)
        vaug_ref[...] = jnp.where(acol == DH, 1.0, 0.0).astype(jnp.bfloat16)

        wq_buf[my] = wq_ref[...]
        wo_buf[my] = wo_ref[...]

        x2 = x_ref[...].reshape(B * SQL, DM)

        def rcopy(buf, s, j, nbr):
            return pltpu.make_async_remote_copy(
                src_ref=buf.at[j], dst_ref=buf.at[j],
                send_sem=send_sems.at[s, j], recv_sem=recv_sems.at[s, j],
                device_id=(nbr,), device_id_type=pl.DeviceIdType.MESH,
            )

        def issue_kv(j, slot):
            ds = []
            for b in range(B):
                for hh in range(H_LOC):
                    g = j * H_LOC + hh
                    idx = b * H_LOC + hh
                    for src, scr in ((k_ref, kscr), (v_ref, vscr)):
                        c = pltpu.make_async_copy(
                            src.at[b, :, g, :], scr.at[slot, idx],
                            dma_sems.at[slot])
                        c.start()
                        ds.append(c)
            return ds

        pending = issue_kv(my, 0)

        sends = []
        for h in range(N_DEV):
            j = lax.rem(my - h + N_DEV, N_DEV)
            slot = h % 2

            if h > 0:
                rcopy(wq_buf, 0, j, left).wait_recv()
            if h < N_DEV - 1:
                s = rcopy(wq_buf, 0, j, right)
                s.start()
                sends.append(s)
                jn = lax.rem(my - h - 1 + N_DEV, N_DEV)
                nxt = issue_kv(jn, 1 - slot)

            for c in pending:
                c.wait()
            pending = nxt if h < N_DEV - 1 else []

            qflat = jnp.dot(x2, wq_buf[j],
                            preferred_element_type=jnp.float32)
            qflat = qflat.astype(jnp.bfloat16)

            for b in range(B):
                for hh in range(H_LOC):
                    idx = b * H_LOC + hh
                    q = qflat[b * SQL:(b + 1) * SQL,
                              hh * DH:(hh + 1) * DH]
                    k = kscr[slot, idx].astype(jnp.bfloat16)
                    sc = lax.dot_general(
                        q, k, (((1,), (1,)), ((), ())),
                        preferred_element_type=jnp.float32)
                    w = jnp.exp(sc + bias_ref[...]).astype(jnp.bfloat16)
                    vaug_ref[:, :DH] = vscr[slot, idx].astype(jnp.bfloat16)
                    cs = jnp.dot(w, vaug_ref[...],
                                 preferred_element_type=jnp.float32)
                    ctx = cs[:, :DH] * (1.0 / cs[:, DH:DH + 1])
                    ctx_ref[b, :, hh * DH:(hh + 1) * DH] = (
                        ctx.astype(jnp.bfloat16))

            if h > 0:
                rcopy(wo_buf, 1, j, left).wait_recv()
            if h < N_DEV - 1:
                s = rcopy(wo_buf, 1, j, right)
                s.start()
                sends.append(s)

            for b in range(B):
                contrib = jnp.dot(ctx_ref[b], wo_buf[j],
                                  preferred_element_type=jnp.float32)
                if h == 0:
                    out_ref[b] = contrib
                else:
                    out_ref[b] += contrib

        for s in sends:
            s.wait_send()

    return pl.pallas_call(
        body,
        out_shape=jax.ShapeDtypeStruct((B, SQL, DM), jnp.float32),
        in_specs=[
            pl.BlockSpec(memory_space=pltpu.VMEM),
            pl.BlockSpec(memory_space=pltpu.VMEM),
            pl.BlockSpec(memory_space=pl.ANY),
            pl.BlockSpec(memory_space=pl.ANY),
            pl.BlockSpec(memory_space=pltpu.VMEM),
        ],
        out_specs=pl.BlockSpec(memory_space=pltpu.VMEM),
        scratch_shapes=[
            pltpu.VMEM((N_DEV, DM, HD_LOC), jnp.bfloat16),
            pltpu.VMEM((N_DEV, HD_LOC, DM), jnp.bfloat16),
            pltpu.VMEM((SQL, SKV), jnp.float32),
            pltpu.VMEM((B, SQL, HD_LOC), jnp.bfloat16),
            pltpu.VMEM((2, B * H_LOC, SKV, DH), jnp.float32),
            pltpu.VMEM((2, B * H_LOC, SKV, DH), jnp.float32),
            pltpu.VMEM((SQL, VA), jnp.bfloat16),
            pltpu.SemaphoreType.DMA((2, N_DEV)),
            pltpu.SemaphoreType.DMA((2, N_DEV)),
            pltpu.SemaphoreType.DMA((2,)),
        ],
        compiler_params=pltpu.CompilerParams(
            collective_id=0,
            vmem_limit_bytes=63 * 1024 * 1024,
        ),
    )(xb, wqb, K_ext, V_ext, wob)


# device time: 179079 ns/iter; 1.1166x vs baseline; 1.1166x over previous
import jax
import jax.numpy as jnp
from jax import lax
from jax.experimental import pallas as pl
from jax.experimental.pallas import tpu as pltpu

N_DEV = 8
B = 2
SQL = 512
DM = 768
SKV = 512
HQ = 64
DH = 64
H_LOC = HQ // N_DEV
HD_LOC = H_LOC * DH
QBLK = 64
VA = 128


def kernel(x, Wq, K_ext, V_ext, Wo):
    xb = (x * 0.125).astype(jnp.bfloat16)
    wqb = Wq.astype(jnp.bfloat16)
    wob = Wo.astype(jnp.bfloat16)
    kb = jnp.transpose(K_ext.astype(jnp.bfloat16), (0, 2, 1, 3))
    vb = jnp.transpose(V_ext.astype(jnp.bfloat16), (0, 2, 1, 3))

    def body(x_ref, wq_ref, k_ref, v_ref, wo_ref, out_ref,
             wq_buf, wo_buf, bias_ref, ctx_ref, vaug_ref,
             send_sems, recv_sems):
        my = lax.axis_index("i")
        left = lax.rem(my - 1 + N_DEV, N_DEV)
        right = lax.rem(my + 1, N_DEV)

        barrier = pltpu.get_barrier_semaphore()
        for nbr in (left, right):
            pl.semaphore_signal(barrier, inc=1, device_id=(nbr,),
                                device_id_type=pl.DeviceIdType.MESH)
        pl.semaphore_wait(barrier, 2)

        row = lax.broadcasted_iota(jnp.int32, (SQL, SKV), 0)
        col = lax.broadcasted_iota(jnp.int32, (SQL, SKV), 1)
        qb = my * (SQL // QBLK) + row // QBLK
        kb_ = col // QBLK
        keep = (qb == kb_) | (kb_ == 0) | (lax.rem(qb + kb_, 3) == 0)
        bias_ref[...] = jnp.where(keep, 0.0, -1e9).astype(jnp.float32)

        acol = lax.broadcasted_iota(jnp.int32, (SQL, VA), 1)
        vaug_ref[...] = jnp.where(acol == DH, 1.0, 0.0).astype(jnp.bfloat16)

        wq_buf[my] = wq_ref[...]
        wo_buf[my] = wo_ref[...]

        x2 = x_ref[...].reshape(B * SQL, DM)

        def rcopy(buf, s, j, nbr):
            return pltpu.make_async_remote_copy(
                src_ref=buf.at[j], dst_ref=buf.at[j],
                send_sem=send_sems.at[s, j], recv_sem=recv_sems.at[s, j],
                device_id=(nbr,), device_id_type=pl.DeviceIdType.MESH,
            )

        sends = []
        for h in range(N_DEV):
            j = lax.rem(my - h + N_DEV, N_DEV)

            if h > 0:
                rcopy(wq_buf, 0, j, left).wait_recv()
            if h < N_DEV - 1:
                s = rcopy(wq_buf, 0, j, right)
                s.start()
                sends.append(s)

            qflat = jnp.dot(x2, wq_buf[j],
                            preferred_element_type=jnp.float32)
            qflat = qflat.astype(jnp.bfloat16)

            for b in range(B):
                for hh in range(H_LOC):
                    q = qflat[b * SQL:(b + 1) * SQL,
                              hh * DH:(hh + 1) * DH]
                    k = k_ref[b, j * H_LOC + hh]
                    sc = lax.dot_general(
                        q, k, (((1,), (1,)), ((), ())),
                        preferred_element_type=jnp.float32)
                    w = jnp.exp(sc + bias_ref[...]).astype(jnp.bfloat16)
                    vaug_ref[:, :DH] = v_ref[b, j * H_LOC + hh]
                    cs = jnp.dot(w, vaug_ref[...],
                                 preferred_element_type=jnp.float32)
                    ctx = cs[:, :DH] * (1.0 / cs[:, DH:DH + 1])
                    ctx_ref[b, :, hh * DH:(hh + 1) * DH] = (
                        ctx.astype(jnp.bfloat16))

            if h > 0:
                rcopy(wo_buf, 1, j, left).wait_recv()
            if h < N_DEV - 1:
                s = rcopy(wo_buf, 1, j, right)
                s.start()
                sends.append(s)

            for b in range(B):
                contrib = jnp.dot(ctx_ref[b], wo_buf[j],
                                  preferred_element_type=jnp.float32)
                if h == 0:
                    out_ref[b] = contrib
                else:
                    out_ref[b] += contrib

        for s in sends:
            s.wait_send()

    return pl.pallas_call(
        body,
        out_shape=jax.ShapeDtypeStruct((B, SQL, DM), jnp.float32),
        in_specs=[pl.BlockSpec(memory_space=pltpu.VMEM)] * 5,
        out_specs=pl.BlockSpec(memory_space=pltpu.VMEM),
        scratch_shapes=[
            pltpu.VMEM((N_DEV, DM, HD_LOC), jnp.bfloat16),
            pltpu.VMEM((N_DEV, HD_LOC, DM), jnp.bfloat16),
            pltpu.VMEM((SQL, SKV), jnp.float32),
            pltpu.VMEM((B, SQL, HD_LOC), jnp.bfloat16),
            pltpu.VMEM((SQL, VA), jnp.bfloat16),
            pltpu.SemaphoreType.DMA((2, N_DEV)),
            pltpu.SemaphoreType.DMA((2, N_DEV)),
        ],
        compiler_params=pltpu.CompilerParams(
            collective_id=0,
            vmem_limit_bytes=63 * 1024 * 1024,
        ),
    )(xb, wqb, kb, vb, wob)
